# baseline (speedup 1.0000x reference)
import functools

import jax
import jax.numpy as jnp
from jax import lax
from jax.experimental import pallas as pl
from jax.experimental.pallas import tpu as pltpu
from jax.experimental.pallas import tpu_sc as plsc

_NE = 8192
_D = 32
_N = 8192
_TN = 1024
_TK = 1024
_COMMIT = 0.25


def _combine_exact(vL, iL, vR, iR):
    keep = vL <= vR
    return jnp.where(keep, vL, vR), jnp.where(keep, iL, iR)


def _combine_rounded(vL, iL, vR, iR):
    vLr = vL.astype(jnp.bfloat16).astype(jnp.float32)
    keep = vLr <= vR
    return jnp.where(keep, vL, vR), jnp.where(keep, iL, iR)


def _argmin_body(flat_ref, w_ref, idx_ref):
    flat = flat_ref[...]
    rowsq = jnp.sum(flat * flat, axis=1, keepdims=True)
    vals = []
    idxs = []
    for k in range(_NE // _TK):
        w = w_ref[pl.ds(k * _TK, _TK), :]
        wsq = jnp.sum(w * w, axis=1)[None, :]
        dot = lax.dot_general(flat, w, (((1,), (1,)), ((), ())),
                              preferred_element_type=jnp.float32)
        dist = (rowsq + wsq) - 2.0 * dot
        cmin = jnp.min(dist, axis=1, keepdims=True)
        cols = lax.broadcasted_iota(jnp.int32, (_TN, _TK), 1)
        carg = jnp.min(jnp.where(dist == cmin, cols, jnp.int32(2**30)),
                       axis=1, keepdims=True) + k * _TK
        vals.append(cmin)
        idxs.append(carg)
    v01, i01 = _combine_exact(vals[0], idxs[0], vals[1], idxs[1])
    v23, i23 = _combine_exact(vals[2], idxs[2], vals[3], idxs[3])
    v45, i45 = _combine_exact(vals[4], idxs[4], vals[5], idxs[5])
    v67, i67 = _combine_exact(vals[6], idxs[6], vals[7], idxs[7])
    vE, iE = _combine_rounded(v01, i01, v23, i23)
    vF, iF = _combine_rounded(v45, i45, v67, i67)
    _, iT = _combine_rounded(vE, iE, vF, iF)
    idx_ref[...] = iT


def _run_argmin(flat, W):
    return pl.pallas_call(
        _argmin_body,
        grid=(_N // _TN,),
        in_specs=[
            pl.BlockSpec((_TN, _D), lambda i: (i, 0)),
            pl.BlockSpec((_NE, _D), lambda i: (0, 0)),
        ],
        out_specs=pl.BlockSpec((_TN, 1), lambda i: (i, 0)),
        out_shape=jax.ShapeDtypeStruct((_N, 1), jnp.int32),
    )(flat, W)


_SC_NW = 32
_SC_ROWS = _N // _SC_NW


@functools.cache
def _make_sc_gather():
    @functools.partial(
        pl.kernel,
        mesh=plsc.VectorSubcoreMesh(core_axis_name="c", subcore_axis_name="s"),
        out_type=jax.ShapeDtypeStruct((_N, _D), jnp.float32),
        scratch_types=[
            pltpu.VMEM((_SC_ROWS,), jnp.int32),
            pltpu.VMEM((_SC_ROWS, _D), jnp.float32),
            pltpu.SemaphoreType.DMA,
        ],
        compiler_params=pltpu.CompilerParams(use_tc_tiling_on_sc=False),
    )
    def _sc_gather(w_hbm, idx_hbm, out_hbm, idx_v, rows_v, sem):
        wid = lax.axis_index("s") * 2 + lax.axis_index("c")
        base = wid * _SC_ROWS
        pltpu.sync_copy(idx_hbm.at[pl.ds(base, _SC_ROWS)], idx_v)
        pltpu.async_copy(w_hbm.at[idx_v], rows_v, sem).wait()
        pltpu.sync_copy(rows_v, out_hbm.at[pl.ds(base, _SC_ROWS)])

    return _sc_gather


def _st_body(flat_ref, q_ref, out_ref, loss_ref):
    i = pl.program_id(0)
    f = flat_ref[...]
    q = q_ref[...]
    d = q - f
    out_ref[...] = f + d

    @pl.when(i == 0)
    def _init():
        loss_ref[...] = jnp.zeros((1, 1), jnp.float32)

    loss_ref[...] += (jnp.sum(d * d) * (_COMMIT / (_N * _D))).reshape(1, 1)


def _run_st(flat, q):
    return pl.pallas_call(
        _st_body,
        grid=(_N // _TN,),
        in_specs=[
            pl.BlockSpec((_TN, _D), lambda i: (i, 0)),
            pl.BlockSpec((_TN, _D), lambda i: (i, 0)),
        ],
        out_specs=[
            pl.BlockSpec((_TN, _D), lambda i: (i, 0)),
            pl.BlockSpec((1, 1), lambda i: (0, 0)),
        ],
        out_shape=[
            jax.ShapeDtypeStruct((_N, _D), jnp.float32),
            jax.ShapeDtypeStruct((1, 1), jnp.float32),
        ],
    )(flat, q)


def kernel(x, W):
    b, c, h, w = x.shape
    flat = jnp.transpose(x, (0, 2, 3, 1)).reshape(-1, c)
    idx2d = _run_argmin(flat, W)
    idx = idx2d.reshape(-1)
    q = _make_sc_gather()(W, idx)
    st, loss = _run_st(flat, q)
    st = st.reshape(b, h, w, c)
    e_loss = loss.reshape(())
    return (e_loss, jnp.transpose(st, (0, 3, 1, 2)), idx.reshape(b, h * w))


# speedup vs baseline: 1.0134x; 1.0134x over previous
import functools

import jax
import jax.numpy as jnp
from jax import lax
from jax.experimental import pallas as pl
from jax.experimental.pallas import tpu as pltpu
from jax.experimental.pallas import tpu_sc as plsc

_NE = 8192
_D = 32
_N = 8192
_TN = 1024
_TK = 1024
_COMMIT = 0.25


def _combine_exact(vL, iL, vR, iR):
    keep = vL <= vR
    return jnp.where(keep, vL, vR), jnp.where(keep, iL, iR)


def _combine_rounded(vL, iL, vR, iR):
    vLr = vL.astype(jnp.bfloat16).astype(jnp.float32)
    keep = vLr <= vR
    return jnp.where(keep, vL, vR), jnp.where(keep, iL, iR)


def _argmin_body(flat_ref, w_ref, idx_ref):
    flat = flat_ref[...]
    rowsq = jnp.sum(flat * flat, axis=1, keepdims=True)
    cols = lax.broadcasted_iota(jnp.int32, (_TN, _TK), 1)
    vals = []
    idxs = []
    for k in range(_NE // _TK):
        w = w_ref[pl.ds(k * _TK, _TK), :]
        wsq = jnp.sum(w * w, axis=1)[None, :]
        dot = lax.dot_general(flat, w, (((1,), (1,)), ((), ())),
                              preferred_element_type=jnp.float32)
        dist = (rowsq + wsq) - 2.0 * dot
        cmin = jnp.min(dist, axis=1, keepdims=True)
        carg = jnp.min(jnp.where(dist == cmin, cols, jnp.int32(2**30)),
                       axis=1, keepdims=True) + k * _TK
        vals.append(cmin)
        idxs.append(carg)
    v01, i01 = _combine_exact(vals[0], idxs[0], vals[1], idxs[1])
    v23, i23 = _combine_exact(vals[2], idxs[2], vals[3], idxs[3])
    v45, i45 = _combine_exact(vals[4], idxs[4], vals[5], idxs[5])
    v67, i67 = _combine_exact(vals[6], idxs[6], vals[7], idxs[7])
    vE, iE = _combine_rounded(v01, i01, v23, i23)
    vF, iF = _combine_rounded(v45, i45, v67, i67)
    _, iT = _combine_rounded(vE, iE, vF, iF)
    idx_ref[...] = iT


def _run_argmin(flat, W):
    return pl.pallas_call(
        _argmin_body,
        grid=(_N // _TN,),
        in_specs=[
            pl.BlockSpec((_TN, _D), lambda i: (i, 0)),
            pl.BlockSpec((_NE, _D), lambda i: (0, 0)),
        ],
        out_specs=pl.BlockSpec((_TN, 1), lambda i: (i, 0)),
        out_shape=jax.ShapeDtypeStruct((_N, 1), jnp.int32),
    )(flat, W)


_SC_NW = 32
_SC_ROWS = _N // _SC_NW


@functools.cache
def _make_sc_gather():
    @functools.partial(
        pl.kernel,
        mesh=plsc.VectorSubcoreMesh(core_axis_name="c", subcore_axis_name="s"),
        out_type=jax.ShapeDtypeStruct((_N, _D), jnp.float32),
        scratch_types=[
            pltpu.VMEM((_SC_ROWS,), jnp.int32),
            pltpu.VMEM((_SC_ROWS, _D), jnp.float32),
            pltpu.SemaphoreType.DMA,
        ],
        compiler_params=pltpu.CompilerParams(use_tc_tiling_on_sc=False),
    )
    def _sc_gather(w_hbm, idx_hbm, out_hbm, idx_v, rows_v, sem):
        wid = lax.axis_index("s") * 2 + lax.axis_index("c")
        base = wid * _SC_ROWS
        pltpu.sync_copy(idx_hbm.at[pl.ds(base, _SC_ROWS)], idx_v)
        pltpu.async_copy(w_hbm.at[idx_v], rows_v, sem).wait()
        pltpu.sync_copy(rows_v, out_hbm.at[pl.ds(base, _SC_ROWS)])

    return _sc_gather


def _st_body(flat_ref, q_ref, out_ref, loss_ref):
    f = flat_ref[...]
    q = q_ref[...]
    d = q - f
    out_ref[...] = f + d
    loss_ref[...] = (jnp.sum(d * d) * (_COMMIT / (_N * _D))).reshape(1, 1)


def _run_st(flat, q):
    return pl.pallas_call(
        _st_body,
        in_specs=[
            pl.BlockSpec((_N, _D), lambda: (0, 0)),
            pl.BlockSpec((_N, _D), lambda: (0, 0)),
        ],
        out_specs=[
            pl.BlockSpec((_N, _D), lambda: (0, 0)),
            pl.BlockSpec((1, 1), lambda: (0, 0)),
        ],
        out_shape=[
            jax.ShapeDtypeStruct((_N, _D), jnp.float32),
            jax.ShapeDtypeStruct((1, 1), jnp.float32),
        ],
    )(flat, q)


def kernel(x, W):
    b, c, h, w = x.shape
    flat = jnp.transpose(x, (0, 2, 3, 1)).reshape(-1, c)
    idx2d = _run_argmin(flat, W)
    idx = idx2d.reshape(-1)
    q = _make_sc_gather()(W, idx)
    st, loss = _run_st(flat, q)
    st = st.reshape(b, h, w, c)
    e_loss = loss.reshape(())
    return (e_loss, jnp.transpose(st, (0, 3, 1, 2)), idx.reshape(b, h * w))
